# baseline (speedup 1.0000x reference)
import functools

import jax
import jax.numpy as jnp
from jax import lax
from jax.experimental import pallas as pl
from jax.experimental.pallas import tpu as pltpu
from jax.experimental.pallas import tpu_sc as plsc

RING_LO = 4
RING_HI = 103
NSLOT = 112
B, S, E, H = 16, 2048, 256, 64
SCALE = H ** -0.5
LANES = 16
TPL = S // LANES


def _sc_body(x2_hbm, seq_hbm, xg_hbm, valid_hbm,
             seq_v, selp_v, sel_abs_v, validf_v, rows_v, sem):
    nc = 2
    wid = lax.axis_index("s") * nc + lax.axis_index("c")

    @pl.when(wid < B)
    def _():
        b = wid
        lane = lax.iota(jnp.int32, LANES)
        pltpu.sync_copy(seq_hbm.at[b], seq_v)

        for i in range(LANES * NSLOT // LANES):
            selp_v[pl.ds(i * LANES, LANES)] = jnp.full((LANES,), -1, jnp.int32)

        def step(t, carry):
            pos = lane * TPL + t
            v = plsc.load_gather(seq_v, [pos])
            ok = (v >= RING_LO) & (v <= RING_HI)
            slot = jnp.where(ok, v - RING_LO, 0)
            tgt = lane * NSLOT + slot
            plsc.store_scatter(selp_v, [tgt], pos, mask=ok)
            return carry

        lax.fori_loop(0, TPL, step, 0, unroll=4)

        base = b * S
        for jc in range(NSLOT // LANES):
            acc = jnp.full((LANES,), -1, jnp.int32)
            for l in range(LANES):
                acc = jnp.maximum(acc, selp_v[pl.ds(l * NSLOT + jc * LANES, LANES)])
            validf_v[pl.ds(jc * LANES, LANES)] = (acc >= 0).astype(jnp.float32)
            sel_abs_v[pl.ds(jc * LANES, LANES)] = jnp.maximum(acc, 0) + base

        pltpu.async_copy(x2_hbm.at[sel_abs_v], rows_v, sem).wait()
        pltpu.sync_copy(rows_v, xg_hbm.at[b])
        pltpu.sync_copy(validf_v, valid_hbm.at[b])


def _sc_select_gather(x2, seq):
    mesh = plsc.VectorSubcoreMesh(core_axis_name="c", subcore_axis_name="s")
    k = functools.partial(
        pl.kernel,
        mesh=mesh,
        compiler_params=pltpu.CompilerParams(needs_layout_passes=False),
        out_type=[
            jax.ShapeDtypeStruct((B, NSLOT, E), jnp.float32),
            jax.ShapeDtypeStruct((B, NSLOT), jnp.float32),
        ],
        scratch_types=[
            pltpu.VMEM((S,), jnp.int32),
            pltpu.VMEM((LANES * NSLOT,), jnp.int32),
            pltpu.VMEM((NSLOT,), jnp.int32),
            pltpu.VMEM((NSLOT,), jnp.float32),
            pltpu.VMEM((NSLOT, E), jnp.float32),
            pltpu.SemaphoreType.DMA,
        ],
    )(_sc_body)
    return k(x2, seq)


def _tc_body(x_ref, xg_ref, valid_ref, w0_ref, b0_ref, w1_ref, b1_ref, out_ref):
    g = jnp.dot(xg_ref[0], w1_ref[...], preferred_element_type=jnp.float32)
    g = g + b1_ref[...]
    out0 = jnp.dot(x_ref[0], w0_ref[...], preferred_element_type=jnp.float32)
    out0 = out0 + b0_ref[...]
    logits = lax.dot_general(
        out0, g, (((1,), (1,)), ((), ())),
        preferred_element_type=jnp.float32)
    logits = logits * valid_ref[0]
    out_ref[0] = SCALE * logits[:, :100]


def kernel(x, sequences, W0, b0, W1, b1):
    x2 = x.reshape(B * S, E)
    xg, valid = _sc_select_gather(x2, sequences)
    valid3 = valid.reshape(B, 1, NSLOT)
    b0r = b0.reshape(1, H)
    b1r = b1.reshape(1, H)
    return pl.pallas_call(
        _tc_body,
        grid=(B,),
        in_specs=[
            pl.BlockSpec((1, S, E), lambda b: (b, 0, 0)),
            pl.BlockSpec((1, NSLOT, E), lambda b: (b, 0, 0)),
            pl.BlockSpec((1, 1, NSLOT), lambda b: (b, 0, 0)),
            pl.BlockSpec((E, H), lambda b: (0, 0)),
            pl.BlockSpec((1, H), lambda b: (0, 0)),
            pl.BlockSpec((E, H), lambda b: (0, 0)),
            pl.BlockSpec((1, H), lambda b: (0, 0)),
        ],
        out_specs=pl.BlockSpec((1, S, 100), lambda b: (b, 0, 0)),
        out_shape=jax.ShapeDtypeStruct((B, S, 100), jnp.float32),
    )(x, xg, valid3, W0, b0r, W1, b1r)
